# Initial kernel scaffold; baseline (speedup 1.0000x reference)
import jax
import jax.numpy as jnp
from jax.experimental import pallas as pl


def kernel(hidden_states, router_W, router_b, W1, W2):
    raise NotImplementedError("write your pallas kernel here")


# speedup vs baseline: 2.4882x; 2.4882x over previous
import functools

import jax
import jax.numpy as jnp
from jax import lax
from jax.experimental import pallas as pl
from jax.experimental.pallas import tpu as pltpu
from jax.experimental.pallas import tpu_sc as plsc

S = 2048
D = 1024
F = 4096
E = 8
EP = 128

T = 256
NT = S // T
G = NT + E - 1
FK = 1024
K = F // FK

NC = 2
NS = 16
NW = NC * NS
CH = S // NW


def _router_body(h_ref, w_ref, b_ref, p_ref, prob_ref, cnt_ref):
    h = h_ref[...]
    logits = jnp.dot(h.astype(jnp.bfloat16), w_ref[...].astype(jnp.bfloat16),
                     preferred_element_type=jnp.float32)
    logits = logits + b_ref[...]
    m = jnp.max(logits, axis=1, keepdims=True)
    z = jnp.exp(logits - m)
    s = jnp.sum(z, axis=1, keepdims=True)
    prob_ref[...] = 1.0 / s[:, 0]

    lane = lax.broadcasted_iota(jnp.int32, (S, EP), 1)
    ind = jnp.min(jnp.where(logits == m, lane, EP), axis=1)
    onehot = (lane == ind[:, None]).astype(jnp.int32)

    c = onehot
    sh = 1
    while sh < S:
        c = c + jnp.concatenate(
            [jnp.zeros((sh, EP), jnp.int32), c[: S - sh]], axis=0)
        sh *= 2
    counts = c[S - 1 : S, :]

    r = lax.broadcasted_iota(jnp.int32, (EP, EP), 0)
    col = lax.broadcasted_iota(jnp.int32, (EP, EP), 1)
    lt = (r < col).astype(jnp.float32)
    off = jnp.dot(counts.astype(jnp.float32), lt,
                  preferred_element_type=jnp.float32,
                  precision=lax.Precision.HIGHEST)


    onef = onehot.astype(jnp.float32)
    rank = jnp.sum(onef * c.astype(jnp.float32), axis=1) - 1.0
    pos = jnp.sum(onef * off, axis=1) + rank
    p_ref[...] = pos.astype(jnp.int32)
    cnt_ref[...] = counts


def _router_call(h, wp, bp):
    return pl.pallas_call(
        _router_body,
        out_shape=(
            jax.ShapeDtypeStruct((S,), jnp.int32),
            jax.ShapeDtypeStruct((S,), jnp.float32),
            jax.ShapeDtypeStruct((1, EP), jnp.int32),
        ),
    )(h, wp, bp)


def _moe_body(tid_ref, eid_ref, lo_ref, hi_ref, first_ref,
              x_ref, w1v_ref, w1g_ref, w2_ref, prob_ref, out_ref):
    w = pl.program_id(0)
    k = pl.program_id(1)
    row0 = tid_ref[w] * T
    rows = row0 + lax.broadcasted_iota(jnp.int32, (T, 1), 0)
    msk = (rows >= lo_ref[w]) & (rows < hi_ref[w])
    x = jnp.where(msk, x_ref[...], 0.0)
    v = jnp.dot(x, w1v_ref[0], preferred_element_type=jnp.float32)
    g = jnp.dot(x, w1g_ref[0], preferred_element_type=jnp.float32)
    a = (v * jax.nn.sigmoid(v)) * g
    y = jnp.dot(a, w2_ref[0], preferred_element_type=jnp.float32)
    y = y * prob_ref[:, 0:1]

    init = jnp.logical_and(k == 0, first_ref[w] == 1)

    @pl.when(init)
    def _():
        out_ref[...] = y

    @pl.when(jnp.logical_not(init))
    def _():
        out_ref[...] = out_ref[...] + y


def _moe_call(tid, eid, lo, hi, first, h_sorted, w1, w2, prob_sorted):
    grid_spec = pltpu.PrefetchScalarGridSpec(
        num_scalar_prefetch=5,
        grid=(G, K),
        in_specs=[
            pl.BlockSpec((T, D), lambda w, k, tid, eid, lo, hi, first: (tid[w], 0)),
            pl.BlockSpec((1, D, FK), lambda w, k, tid, eid, lo, hi, first: (eid[w], 0, k)),
            pl.BlockSpec((1, D, FK), lambda w, k, tid, eid, lo, hi, first: (eid[w], 0, K + k)),
            pl.BlockSpec((1, FK, D), lambda w, k, tid, eid, lo, hi, first: (eid[w], k, 0)),
            pl.BlockSpec((T, EP), lambda w, k, tid, eid, lo, hi, first: (tid[w], 0)),
        ],
        out_specs=pl.BlockSpec((T, D), lambda w, k, tid, eid, lo, hi, first: (tid[w], 0)),
    )
    return pl.pallas_call(
        _moe_body,
        grid_spec=grid_spec,
        out_shape=jax.ShapeDtypeStruct((S, D), jnp.float32),
        compiler_params=pltpu.CompilerParams(
            dimension_semantics=("arbitrary", "arbitrary")),
    )(tid, eid, lo, hi, first, h_sorted, w1, w1, w2, prob_sorted)


@functools.cache
def _sc_kernels():
    mesh = plsc.VectorSubcoreMesh(core_axis_name="c", subcore_axis_name="s")

    @functools.partial(
        pl.kernel,
        mesh=mesh,
        out_type=(
            jax.ShapeDtypeStruct((S, D), jnp.float32),
            jax.ShapeDtypeStruct((S, EP), jnp.float32),
        ),
        scratch_types=[
            pltpu.VMEM((CH,), jnp.int32),
            pltpu.VMEM((CH, D), jnp.float32),
            pltpu.VMEM((CH, EP), jnp.float32),
            pltpu.SemaphoreType.DMA,
            pltpu.SemaphoreType.DMA,
        ],
    )
    def sc_scatter(h_hbm, probp_hbm, p_hbm, hs_hbm, ps_hbm,
                   idx_v, rows_v, prob_v, sem_a, sem_b):
        wid = lax.axis_index("s") * NC + lax.axis_index("c")
        base = wid * CH
        pltpu.sync_copy(p_hbm.at[pl.ds(base, CH)], idx_v)
        pltpu.sync_copy(h_hbm.at[pl.ds(base, CH)], rows_v)
        pltpu.sync_copy(probp_hbm.at[pl.ds(base, CH)], prob_v)
        cp_a = pltpu.async_copy(rows_v, hs_hbm.at[idx_v], sem_a)
        cp_b = pltpu.async_copy(prob_v, ps_hbm.at[idx_v], sem_b)
        cp_a.wait()
        cp_b.wait()

    @functools.partial(
        pl.kernel,
        mesh=mesh,
        out_type=jax.ShapeDtypeStruct((S, D), jnp.float32),
        scratch_types=[
            pltpu.VMEM((CH,), jnp.int32),
            pltpu.VMEM((CH, D), jnp.float32),
            pltpu.SemaphoreType.DMA,
        ],
    )
    def sc_gather(ys_hbm, p_hbm, out_hbm, idx_v, rows_v, sem):
        wid = lax.axis_index("s") * NC + lax.axis_index("c")
        base = wid * CH
        pltpu.sync_copy(p_hbm.at[pl.ds(base, CH)], idx_v)
        pltpu.async_copy(ys_hbm.at[idx_v], rows_v, sem).wait()
        pltpu.sync_copy(rows_v, out_hbm.at[pl.ds(base, CH)])

    return sc_scatter, sc_gather


def _sc_scatter_call(h, probp, p):
    return _sc_kernels()[0](h, probp, p)


def _sc_gather_call(ys, p):
    return _sc_kernels()[1](ys, p)


def _segments(counts):
    csum = jnp.cumsum(counts)
    off = jnp.concatenate([jnp.zeros((1,), jnp.int32), csum[:-1]])
    off_full = jnp.concatenate([off, jnp.array([S], jnp.int32)])
    cuts = jnp.sort(jnp.concatenate(
        [jnp.arange(1, NT, dtype=jnp.int32) * T, off[1:]]))
    starts = jnp.concatenate([jnp.zeros((1,), jnp.int32), cuts])
    ends = jnp.concatenate([cuts, jnp.array([S], jnp.int32)])
    tid = jnp.clip(starts // T, 0, NT - 1)
    eid = jnp.clip(
        jnp.searchsorted(off_full, starts, side="right").astype(jnp.int32) - 1,
        0, E - 1)
    first = jnp.concatenate(
        [jnp.ones((1,), jnp.int32), (tid[1:] != tid[:-1]).astype(jnp.int32)])
    return tid, eid, starts, ends, first


def kernel(hidden_states, router_W, router_b, W1, W2):
    h = hidden_states.reshape(S, D).astype(jnp.float32)
    wp = jnp.zeros((D, EP), jnp.float32).at[:, :E].set(router_W)
    bp = jnp.full((1, EP), -1e30, jnp.float32).at[0, :E].set(router_b)

    p, prob, cnt = _router_call(h, wp, bp)
    counts = cnt[0, :E].astype(jnp.int32)
    tid, eid, lo, hi, first = _segments(counts)

    probp = jnp.zeros((S, EP), jnp.float32).at[:, 0].set(prob)
    h_sorted, prob_sorted = _sc_scatter_call(h, probp, p)
    y_sorted = _moe_call(tid, eid, lo, hi, first, h_sorted, W1, W2, prob_sorted)
    y = _sc_gather_call(y_sorted, p)
    return y.reshape(hidden_states.shape)
